# baseline (device time: 258762 ns/iter reference)
import jax
import jax.numpy as jnp
from jax import lax
from jax.experimental import pallas as pl
from jax.experimental.pallas import tpu as pltpu

N_DEV = 32
SQ = 512
D = 1024
DH = 128
HQ_PER = 8
KV_PER = 2
CHUNK = SQ // N_DEV
SCALE = 0.08838834764831843
N_STEPS = 2 * (N_DEV - 1)


def kernel(x, Wq, Wo, Wk, Wv):
    my = lax.axis_index("i")
    wk_my = lax.dynamic_slice_in_dim(Wk, my * (KV_PER * DH), KV_PER * DH, axis=1)
    wv_my = lax.dynamic_slice_in_dim(Wv, my * (KV_PER * DH), KV_PER * DH, axis=1)

    def body(x_ref, wq_ref, wo_ref, wk_ref, wv_ref, out_ref,
             attn_ref, send_buf, recv_buf, send_sems, recv_sems, credit_sem):
        my_pos = lax.axis_index("i")
        left = (my_pos - 1) % N_DEV
        right = (my_pos + 1) % N_DEV

        xm = x_ref[0]
        q = jnp.dot(xm, wq_ref[...], preferred_element_type=jnp.float32)
        k = jnp.dot(xm, wk_ref[...], preferred_element_type=jnp.float32)
        v = jnp.dot(xm, wv_ref[...], preferred_element_type=jnp.float32)
        for j in range(HQ_PER):
            g = j // 4
            qh = q[:, j * DH:(j + 1) * DH]
            kh = k[:, g * DH:(g + 1) * DH]
            vh = v[:, g * DH:(g + 1) * DH]
            s = lax.dot_general(qh, kh, (((1,), (1,)), ((), ())),
                                preferred_element_type=jnp.float32) * SCALE
            m = jnp.max(s, axis=1, keepdims=True)
            p = jnp.exp(s - m)
            l = jnp.sum(p, axis=1, keepdims=True)
            oh = jnp.dot(p, vh, preferred_element_type=jnp.float32) / l
            attn_ref[:, j * DH:(j + 1) * DH] = oh
        out_ref[0] = jnp.dot(attn_ref[...], wo_ref[...],
                             preferred_element_type=jnp.float32)

        barrier = pltpu.get_barrier_semaphore()
        for nbr in (left, right):
            pl.semaphore_signal(barrier, inc=1, device_id=(nbr,),
                                device_id_type=pl.DeviceIdType.MESH)
        pl.semaphore_wait(barrier, 2)

        for s_i in range(N_STEPS):
            slot = s_i % 2
            if s_i < N_DEV - 1:
                send_c = (my_pos - s_i) % N_DEV
                recv_c = (my_pos - s_i - 1) % N_DEV
            else:
                t = s_i - (N_DEV - 1)
                send_c = (my_pos + 1 - t) % N_DEV
                recv_c = (my_pos - t) % N_DEV
            send_buf[slot] = out_ref[0, pl.ds(send_c * CHUNK, CHUNK), :]
            if s_i >= 2:
                pl.semaphore_wait(credit_sem, 1)
            rdma = pltpu.make_async_remote_copy(
                src_ref=send_buf.at[slot],
                dst_ref=recv_buf.at[slot],
                send_sem=send_sems.at[slot],
                recv_sem=recv_sems.at[slot],
                device_id=(right,),
                device_id_type=pl.DeviceIdType.MESH,
            )
            rdma.start()
            rdma.wait()
            if s_i < N_DEV - 1:
                out_ref[0, pl.ds(recv_c * CHUNK, CHUNK), :] += recv_buf[slot]
            else:
                out_ref[0, pl.ds(recv_c * CHUNK, CHUNK), :] = recv_buf[slot]
            if s_i < N_STEPS - 2:
                pl.semaphore_signal(credit_sem, inc=1, device_id=(left,),
                                    device_id_type=pl.DeviceIdType.MESH)

    return pl.pallas_call(
        body,
        out_shape=jax.ShapeDtypeStruct((1, SQ, D), jnp.float32),
        in_specs=[pl.BlockSpec(memory_space=pltpu.VMEM)] * 5,
        out_specs=pl.BlockSpec(memory_space=pltpu.VMEM),
        scratch_shapes=[
            pltpu.VMEM((SQ, HQ_PER * DH), jnp.float32),
            pltpu.VMEM((2, CHUNK, D), jnp.float32),
            pltpu.VMEM((2, CHUNK, D), jnp.float32),
            pltpu.SemaphoreType.DMA((2,)),
            pltpu.SemaphoreType.DMA((2,)),
            pltpu.SemaphoreType.REGULAR,
        ],
        compiler_params=pltpu.CompilerParams(collective_id=0),
    )(x, Wq, Wo, wk_my, wv_my)
